# baseline (speedup 1.0000x reference)
import functools

import jax
import jax.numpy as jnp
from jax import lax
from jax.experimental import pallas as pl
from jax.experimental.pallas import tpu as pltpu
from jax.experimental.pallas import tpu_sc as plsc

VOCAB = 100000
DIM = 64
DP = DIM // 2
B = 16384
NEG = 5
CK = 1 + NEG

TBLK = 4096
NBLK = (VOCAB + TBLK - 1) // TBLK
VPAD = NBLK * TBLK

_info = plsc.get_sparse_core_info()
NC, NS, L = _info.num_cores, _info.num_subcores, _info.num_lanes
NW = NC * NS
BPW = B // NW
CHUNK = 128
NCH = BPW // CHUNK
GROUPS = CHUNK // 16

_mesh = plsc.VectorSubcoreMesh(core_axis_name="c", subcore_axis_name="s")


def _pack_table(w_ref, out_ref):
    r = lax.broadcasted_iota(jnp.int32, (DIM, DP), 0)
    c = lax.broadcasted_iota(jnp.int32, (DIM, DP), 1)
    sel_e = jnp.where(r == 2 * c, 1.0, 0.0)
    sel_o = jnp.where(r == 2 * c + 1, 1.0, 0.0)
    dn = (((0,), (0,)), ((), ()))
    te = lax.dot_general(w_ref[...], sel_e, dn,
                         preferred_element_type=jnp.float32)
    to = lax.dot_general(w_ref[...], sel_o, dn,
                         preferred_element_type=jnp.float32)
    ue = lax.bitcast_convert_type(te, jnp.uint32)
    uo = lax.bitcast_convert_type(to, jnp.uint32)
    packed = ((ue + jnp.uint32(0x8000)) >> 16) | \
        ((uo + jnp.uint32(0x8000)) & jnp.uint32(0xFFFF0000))
    pk = lax.bitcast_convert_type(packed, jnp.float32)
    q = TBLK // 4
    for j in range(4):
        out_ref[:, j * DP:(j + 1) * DP] = pk[j * q:(j + 1) * q]


def _repack_body(win_ref, wout_ref, pin_ref, pout_ref):
    _pack_table(win_ref, pin_ref)
    _pack_table(wout_ref, pout_ref)


_repack = pl.pallas_call(
    _repack_body,
    grid=(NBLK,),
    in_specs=[
        pl.BlockSpec((DIM, TBLK), lambda i: (0, i)),
        pl.BlockSpec((DIM, TBLK), lambda i: (0, i)),
    ],
    out_specs=[
        pl.BlockSpec((TBLK // 4, 128), lambda i: (i, 0)),
        pl.BlockSpec((TBLK // 4, 128), lambda i: (i, 0)),
    ],
    out_shape=[
        jax.ShapeDtypeStruct((VPAD // 4, 128), jnp.float32),
        jax.ShapeDtypeStruct((VPAD // 4, 128), jnp.float32),
    ],
)

_QSH = (TBLK // 4).bit_length() - 1


@functools.partial(
    pl.kernel,
    out_type=jax.ShapeDtypeStruct((B * CK,), jnp.float32),
    mesh=_mesh,
    compiler_params=pltpu.CompilerParams(
        needs_layout_passes=False, use_tc_tiling_on_sc=False),
    scratch_types=[
        pltpu.VMEM((B // NW // 128, 128), jnp.int32),
        pltpu.VMEM((B // NW // 128, 128), jnp.int32),
        pltpu.VMEM((B * NEG // NW // 128, 128), jnp.int32),
        pltpu.VMEM((CHUNK, DP), jnp.float32),
        pltpu.VMEM((CHUNK, DP), jnp.float32),
        pltpu.VMEM((CHUNK * CK, DP), jnp.float32),
        pltpu.VMEM((CHUNK * CK, DP), jnp.float32),
        pltpu.VMEM((BPW * CK,), jnp.float32),
        pltpu.SemaphoreType.DMA,
        pltpu.SemaphoreType.DMA,
    ],
)
def _sc_logits(cen_hbm, ctx_hbm, neg_hbm, w_in_hbm, w_out_hbm, out_hbm,
               cidx, xidx, nidx, crow0, crow1, prow0, prow1, logit_v,
               sem0, sem1):
    wid = lax.axis_index("s") * NC + lax.axis_index("c")
    crows = (crow0, crow1)
    prows = (prow0, prow1)
    sems = (sem0, sem1)

    pltpu.sync_copy(cen_hbm.at[pl.ds(wid * NCH, NCH)], cidx)
    pltpu.sync_copy(ctx_hbm.at[pl.ds(wid * NCH, NCH)], xidx)
    for k in range(NEG):
        pltpu.sync_copy(neg_hbm.at[pl.ds(k * (B // 128) + wid * NCH, NCH)],
                        nidx.at[pl.ds(k * NCH, NCH)])

    for ref, rows in ((cidx, NCH), (xidx, NCH), (nidx, NCH * NEG)):
        for r in range(rows):
            for j in range(128 // 16):
                v = ref[r, pl.ds(j * 16, 16)]
                ref[r, pl.ds(j * 16, 16)] = (
                    jnp.bitwise_and(v, -TBLK)
                    + ((v & (TBLK // 4 - 1)) << 2)
                    + ((v >> _QSH) & 3))

    def issue(ch):
        bi = ch % 2
        sem = sems[bi]
        cp = [pltpu.async_copy(w_in_hbm.at[cidx.at[ch]], crows[bi], sem),
              pltpu.async_copy(w_out_hbm.at[xidx.at[ch]],
                               prows[bi].at[pl.ds(0, CHUNK)], sem)]
        for k in range(NEG):
            cp.append(pltpu.async_copy(
                w_out_hbm.at[nidx.at[k * NCH + ch]],
                prows[bi].at[pl.ds(CHUNK + k * CHUNK, CHUNK)], sem))
        return cp

    himask = jnp.broadcast_to(jnp.uint32(0xFFFF0000), (16,))
    iota16 = lax.iota(jnp.int32, 16)
    pending = issue(0)
    for ch in range(NCH):
        nxt = issue(ch + 1) if ch + 1 < NCH else []
        for c in pending:
            c.wait()
        pending = nxt
        bi = ch % 2
        crow, prow = crows[bi], prows[bi]

        def group_body(g, _, ch=ch, crow=crow, prow=prow):
            bvec = iota16 + g * 16
            rowp = [bvec] + [CHUNK + k * CHUNK + bvec for k in range(NEG)]

            def d_block(db, accs):
                out = list(accs)
                for dd in range(8):
                    dvec = jnp.bitwise_and(iota16 + (db * 8 + dd), DP - 1)
                    cw = plsc.bitcast(plsc.load_gather(crow, [bvec, dvec]),
                                      jnp.uint32)
                    ce = plsc.bitcast(cw << 16, jnp.float32)
                    co = plsc.bitcast(cw & himask, jnp.float32)
                    for p in range(CK):
                        xw = plsc.bitcast(
                            plsc.load_gather(prow, [rowp[p], dvec]),
                            jnp.uint32)
                        xe = plsc.bitcast(xw << 16, jnp.float32)
                        xo = plsc.bitcast(xw & himask, jnp.float32)
                        out[p] = out[p] + ce * xe + co * xo
                return tuple(out)

            accs = lax.fori_loop(
                0, DP // 8, d_block,
                tuple(jnp.zeros((16,), jnp.float32) for _ in range(CK)))
            flat = (bvec + ch * CHUNK) * CK
            for p in range(CK):
                plsc.store_scatter(logit_v, [flat + p], accs[p])
            return 0

        lax.fori_loop(0, GROUPS, group_body, 0)

    pltpu.sync_copy(logit_v, out_hbm.at[pl.ds(wid * BPW * CK, BPW * CK)])


_NROWS = B * CK // 128


def _bce_body(lg_ref, out_ref):
    x = lg_ref[...]
    r = lax.broadcasted_iota(jnp.int32, (_NROWS, 128), 0)
    c = lax.broadcasted_iota(jnp.int32, (_NROWS, 128), 1)
    pos = (r * 128 + c) % CK
    y = jnp.where(pos == 0, 1.0, 0.0)
    elem = jnp.maximum(x, 0.0) - x * y + jnp.log1p(jnp.exp(-jnp.abs(x)))
    out_ref[...] = (jnp.sum(elem) * (1.0 / (B * CK))).reshape(1, 1)


def kernel(center, context, negatives, W_in, W_out):
    cen = center.astype(jnp.int32).reshape(-1, 128)
    ctx = context.astype(jnp.int32).reshape(-1, 128)
    neg = negatives.T.astype(jnp.int32).reshape(-1, 128)
    pin, pout = _repack(W_in.T, W_out.T)
    logits = _sc_logits(cen, ctx, neg,
                        pin.reshape(VPAD, DP), pout.reshape(VPAD, DP))
    loss = pl.pallas_call(
        _bce_body,
        out_shape=jax.ShapeDtypeStruct((1, 1), jnp.float32),
    )(logits.reshape(_NROWS, 128))
    return loss[0, 0]


# speedup vs baseline: 1.1192x; 1.1192x over previous
import functools

import jax
import jax.numpy as jnp
from jax import lax
from jax.experimental import pallas as pl
from jax.experimental.pallas import tpu as pltpu
from jax.experimental.pallas import tpu_sc as plsc

VOCAB = 100000
DIM = 64
B = 16384
NEG = 5
CK = 1 + NEG

TBLK = 4096
NBLK = (VOCAB + TBLK - 1) // TBLK
VPAD = NBLK * TBLK

_info = plsc.get_sparse_core_info()
NC, NS, L = _info.num_cores, _info.num_subcores, _info.num_lanes
NW = NC * NS
BPW = B // NW
CHUNK = 128
NCH = BPW // CHUNK
GROUPS = CHUNK // 16

_mesh = plsc.VectorSubcoreMesh(core_axis_name="c", subcore_axis_name="s")


def _repack_body(win_ref, wout_ref, pin_ref, pout_ref):
    r = lax.broadcasted_iota(jnp.int32, (DIM, DIM), 0)
    c = lax.broadcasted_iota(jnp.int32, (DIM, DIM), 1)
    eye = jnp.where(r == c, 1.0, 0.0).astype(jnp.bfloat16)
    dn = (((0,), (0,)), ((), ()))
    ti = lax.dot_general(win_ref[...].astype(jnp.bfloat16), eye, dn,
                         preferred_element_type=jnp.float32)
    pin_ref[:, 0:DIM] = ti[:TBLK // 2]
    pin_ref[:, DIM:128] = ti[TBLK // 2:]
    to = lax.dot_general(wout_ref[...].astype(jnp.bfloat16), eye, dn,
                         preferred_element_type=jnp.float32)
    pout_ref[:, 0:DIM] = to[:TBLK // 2]
    pout_ref[:, DIM:128] = to[TBLK // 2:]


_repack = pl.pallas_call(
    _repack_body,
    grid=(NBLK,),
    in_specs=[
        pl.BlockSpec((DIM, TBLK), lambda i: (0, i)),
        pl.BlockSpec((DIM, TBLK), lambda i: (0, i)),
    ],
    out_specs=[
        pl.BlockSpec((TBLK // 2, 128), lambda i: (i, 0)),
        pl.BlockSpec((TBLK // 2, 128), lambda i: (i, 0)),
    ],
    out_shape=[
        jax.ShapeDtypeStruct((VPAD // 2, 128), jnp.float32),
        jax.ShapeDtypeStruct((VPAD // 2, 128), jnp.float32),
    ],
)

_HSH = (TBLK // 2).bit_length() - 1


@functools.partial(
    pl.kernel,
    out_type=jax.ShapeDtypeStruct((B * CK,), jnp.float32),
    mesh=_mesh,
    compiler_params=pltpu.CompilerParams(
        needs_layout_passes=False, use_tc_tiling_on_sc=False),
    scratch_types=[
        pltpu.VMEM((B // NW // 128, 128), jnp.int32),
        pltpu.VMEM((B // NW // 128, 128), jnp.int32),
        pltpu.VMEM((B * NEG // NW // 128, 128), jnp.int32),
        pltpu.VMEM((CHUNK, DIM), jnp.float32),
        pltpu.VMEM((CHUNK, DIM), jnp.float32),
        pltpu.VMEM((CHUNK * CK, DIM), jnp.float32),
        pltpu.VMEM((CHUNK * CK, DIM), jnp.float32),
        pltpu.VMEM((BPW * CK,), jnp.float32),
        pltpu.SemaphoreType.DMA,
        pltpu.SemaphoreType.DMA,
    ],
)
def _sc_logits(cen_hbm, ctx_hbm, neg_hbm, w_in_hbm, w_out_hbm, out_hbm,
               cidx, xidx, nidx, crow0, crow1, prow0, prow1, logit_v,
               sem0, sem1):
    wid = lax.axis_index("s") * NC + lax.axis_index("c")
    crows = (crow0, crow1)
    prows = (prow0, prow1)
    sems = (sem0, sem1)

    pltpu.sync_copy(cen_hbm.at[pl.ds(wid * NCH, NCH)], cidx)
    pltpu.sync_copy(ctx_hbm.at[pl.ds(wid * NCH, NCH)], xidx)
    for k in range(NEG):
        pltpu.sync_copy(neg_hbm.at[pl.ds(k * (B // 128) + wid * NCH, NCH)],
                        nidx.at[pl.ds(k * NCH, NCH)])

    for ref, rows in ((cidx, NCH), (xidx, NCH), (nidx, NCH * NEG)):
        for r in range(rows):
            for j in range(128 // 16):
                v = ref[r, pl.ds(j * 16, 16)]
                ref[r, pl.ds(j * 16, 16)] = (
                    jnp.bitwise_and(v, -TBLK)
                    + ((v & (TBLK // 2 - 1)) << 1)
                    + ((v >> _HSH) & 1))

    def issue(ch):
        bi = ch % 2
        sem = sems[bi]
        cp = [pltpu.async_copy(w_in_hbm.at[cidx.at[ch]], crows[bi], sem),
              pltpu.async_copy(w_out_hbm.at[xidx.at[ch]],
                               prows[bi].at[pl.ds(0, CHUNK)], sem)]
        for k in range(NEG):
            cp.append(pltpu.async_copy(
                w_out_hbm.at[nidx.at[k * NCH + ch]],
                prows[bi].at[pl.ds(CHUNK + k * CHUNK, CHUNK)], sem))
        return cp

    iota16 = lax.iota(jnp.int32, 16)
    pending = issue(0)
    for ch in range(NCH):
        nxt = issue(ch + 1) if ch + 1 < NCH else []
        for c in pending:
            c.wait()
        pending = nxt
        bi = ch % 2
        crow, prow = crows[bi], prows[bi]

        def group_body(g, _, ch=ch, crow=crow, prow=prow):
            bvec = iota16 + g * 16
            rowp = [bvec] + [CHUNK + k * CHUNK + bvec for k in range(NEG)]

            def d_block(db, accs):
                out = list(accs)
                for dd in range(8):
                    dvec = jnp.bitwise_and(iota16 + (db * 8 + dd), DIM - 1)
                    cv = plsc.load_gather(crow, [bvec, dvec])
                    for p in range(CK):
                        xv = plsc.load_gather(prow, [rowp[p], dvec])
                        out[p] = out[p] + cv * xv
                return tuple(out)

            accs = lax.fori_loop(
                0, DIM // 8, d_block,
                tuple(jnp.zeros((16,), jnp.float32) for _ in range(CK)))
            flat = (bvec + ch * CHUNK) * CK
            for p in range(CK):
                plsc.store_scatter(logit_v, [flat + p], accs[p])
            return 0

        lax.fori_loop(0, GROUPS, group_body, 0)

    pltpu.sync_copy(logit_v, out_hbm.at[pl.ds(wid * BPW * CK, BPW * CK)])


_NROWS = B * CK // 128


def _bce_body(lg_ref, out_ref):
    x = lg_ref[...]
    r = lax.broadcasted_iota(jnp.int32, (_NROWS, 128), 0)
    c = lax.broadcasted_iota(jnp.int32, (_NROWS, 128), 1)
    pos = (r * 128 + c) % CK
    y = jnp.where(pos == 0, 1.0, 0.0)
    elem = jnp.maximum(x, 0.0) - x * y + jnp.log1p(jnp.exp(-jnp.abs(x)))
    out_ref[...] = (jnp.sum(elem) * (1.0 / (B * CK))).reshape(1, 1)


def kernel(center, context, negatives, W_in, W_out):
    cen = center.astype(jnp.int32).reshape(-1, 128)
    ctx = context.astype(jnp.int32).reshape(-1, 128)
    neg = negatives.T.astype(jnp.int32).reshape(-1, 128)
    pin, pout = _repack(W_in.T, W_out.T)
    logits = _sc_logits(cen, ctx, neg,
                        pin.reshape(VPAD, DIM), pout.reshape(VPAD, DIM))
    loss = pl.pallas_call(
        _bce_body,
        out_shape=jax.ShapeDtypeStruct((1, 1), jnp.float32),
    )(logits.reshape(_NROWS, 128))
    return loss[0, 0]


# speedup vs baseline: 1.2015x; 1.0735x over previous
import functools

import jax
import jax.numpy as jnp
from jax import lax
from jax.experimental import pallas as pl
from jax.experimental.pallas import tpu as pltpu
from jax.experimental.pallas import tpu_sc as plsc

VOCAB = 100000
DIM = 64
B = 16384
NEG = 5
CK = 1 + NEG

TBLK = 8192
NBLK = (VOCAB + TBLK - 1) // TBLK
VPAD = NBLK * TBLK

_info = plsc.get_sparse_core_info()
NC, NS, L = _info.num_cores, _info.num_subcores, _info.num_lanes
NW = NC * NS
BPW = B // NW
CHUNK = 128
NCH = BPW // CHUNK
GROUPS = CHUNK // 16

_mesh = plsc.VectorSubcoreMesh(core_axis_name="c", subcore_axis_name="s")


def _repack_body(win_ref, wout_ref, pin_ref, pout_ref):
    r = lax.broadcasted_iota(jnp.int32, (DIM, DIM), 0)
    c = lax.broadcasted_iota(jnp.int32, (DIM, DIM), 1)
    eye = jnp.where(r == c, 1.0, 0.0).astype(jnp.bfloat16)
    dn = (((0,), (0,)), ((), ()))
    ti = lax.dot_general(win_ref[...].astype(jnp.bfloat16), eye, dn,
                         preferred_element_type=jnp.float32)
    pin_ref[:, 0:DIM] = ti[:TBLK // 2]
    pin_ref[:, DIM:128] = ti[TBLK // 2:]
    to = lax.dot_general(wout_ref[...].astype(jnp.bfloat16), eye, dn,
                         preferred_element_type=jnp.float32)
    pout_ref[:, 0:DIM] = to[:TBLK // 2]
    pout_ref[:, DIM:128] = to[TBLK // 2:]


_repack = pl.pallas_call(
    _repack_body,
    grid=(NBLK,),
    in_specs=[
        pl.BlockSpec((DIM, TBLK), lambda i: (0, i)),
        pl.BlockSpec((DIM, TBLK), lambda i: (0, i)),
    ],
    out_specs=[
        pl.BlockSpec((TBLK // 2, 128), lambda i: (i, 0)),
        pl.BlockSpec((TBLK // 2, 128), lambda i: (i, 0)),
    ],
    out_shape=[
        jax.ShapeDtypeStruct((VPAD // 2, 128), jnp.float32),
        jax.ShapeDtypeStruct((VPAD // 2, 128), jnp.float32),
    ],
)

_HSH = (TBLK // 2).bit_length() - 1


@functools.partial(
    pl.kernel,
    out_type=jax.ShapeDtypeStruct((B * CK,), jnp.float32),
    mesh=_mesh,
    compiler_params=pltpu.CompilerParams(
        needs_layout_passes=False, use_tc_tiling_on_sc=False),
    scratch_types=[
        pltpu.VMEM((B // NW // 128, 128), jnp.int32),
        pltpu.VMEM((B // NW // 128, 128), jnp.int32),
        pltpu.VMEM((B * NEG // NW // 128, 128), jnp.int32),
        pltpu.VMEM((CHUNK, DIM), jnp.float32),
        pltpu.VMEM((CHUNK, DIM), jnp.float32),
        pltpu.VMEM((CHUNK * CK, DIM), jnp.float32),
        pltpu.VMEM((CHUNK * CK, DIM), jnp.float32),
        pltpu.VMEM((BPW * CK,), jnp.float32),
        pltpu.SemaphoreType.DMA,
        pltpu.SemaphoreType.DMA,
    ],
)
def _sc_logits(cen_hbm, ctx_hbm, neg_hbm, w_in_hbm, w_out_hbm, out_hbm,
               cidx, xidx, nidx, crow0, crow1, prow0, prow1, logit_v,
               sem0, sem1):
    wid = lax.axis_index("s") * NC + lax.axis_index("c")
    crows = (crow0, crow1)
    prows = (prow0, prow1)
    sems = (sem0, sem1)

    pltpu.sync_copy(cen_hbm.at[pl.ds(wid * NCH, NCH)], cidx)
    pltpu.sync_copy(ctx_hbm.at[pl.ds(wid * NCH, NCH)], xidx)
    for k in range(NEG):
        pltpu.sync_copy(neg_hbm.at[pl.ds(k * (B // 128) + wid * NCH, NCH)],
                        nidx.at[pl.ds(k * NCH, NCH)])

    for ref, rows in ((cidx, NCH), (xidx, NCH), (nidx, NCH * NEG)):
        for r in range(rows):
            for j in range(128 // 16):
                v = ref[r, pl.ds(j * 16, 16)]
                ref[r, pl.ds(j * 16, 16)] = (
                    jnp.bitwise_and(v, -TBLK)
                    + ((v & (TBLK // 2 - 1)) << 1)
                    + ((v >> _HSH) & 1))

    def issue(ch):
        bi = ch % 2
        sem = sems[bi]
        cp = [pltpu.async_copy(w_in_hbm.at[cidx.at[ch]], crows[bi], sem),
              pltpu.async_copy(w_out_hbm.at[xidx.at[ch]],
                               prows[bi].at[pl.ds(0, CHUNK)], sem)]
        for k in range(NEG):
            cp.append(pltpu.async_copy(
                w_out_hbm.at[nidx.at[k * NCH + ch]],
                prows[bi].at[pl.ds(CHUNK + k * CHUNK, CHUNK)], sem))
        return cp

    iota16 = lax.iota(jnp.int32, 16)
    pending = issue(0)
    for ch in range(NCH):
        nxt = issue(ch + 1) if ch + 1 < NCH else []
        for c in pending:
            c.wait()
        pending = nxt
        bi = ch % 2
        crow, prow = crows[bi], prows[bi]

        def group_body(g, _, ch=ch, crow=crow, prow=prow):
            bvec = iota16 + g * 16
            rowp = [bvec] + [CHUNK + k * CHUNK + bvec for k in range(NEG)]

            def d_block(db, accs):
                out = list(accs)
                for dd in range(8):
                    dvec = jnp.bitwise_and(iota16 + (db * 8 + dd), DIM - 1)
                    cv = plsc.load_gather(crow, [bvec, dvec])
                    for p in range(CK):
                        xv = plsc.load_gather(prow, [rowp[p], dvec])
                        out[p] = out[p] + cv * xv
                return tuple(out)

            accs = lax.fori_loop(
                0, DIM // 8, d_block,
                tuple(jnp.zeros((16,), jnp.float32) for _ in range(CK)))
            flat = (bvec + ch * CHUNK) * CK
            for p in range(CK):
                plsc.store_scatter(logit_v, [flat + p], accs[p])
            return 0

        lax.fori_loop(0, GROUPS, group_body, 0)

    pltpu.sync_copy(logit_v, out_hbm.at[pl.ds(wid * BPW * CK, BPW * CK)])


_NROWS = B * CK // 128


def _bce_body(lg_ref, out_ref):
    x = lg_ref[...]
    r = lax.broadcasted_iota(jnp.int32, (_NROWS, 128), 0)
    c = lax.broadcasted_iota(jnp.int32, (_NROWS, 128), 1)
    pos = (r * 128 + c) % CK
    y = jnp.where(pos == 0, 1.0, 0.0)
    elem = jnp.maximum(x, 0.0) - x * y + jnp.log1p(jnp.exp(-jnp.abs(x)))
    out_ref[...] = (jnp.sum(elem) * (1.0 / (B * CK))).reshape(1, 1)


def kernel(center, context, negatives, W_in, W_out):
    cen = center.astype(jnp.int32).reshape(-1, 128)
    ctx = context.astype(jnp.int32).reshape(-1, 128)
    neg = negatives.T.astype(jnp.int32).reshape(-1, 128)
    pin, pout = _repack(W_in.T, W_out.T)
    logits = _sc_logits(cen, ctx, neg,
                        pin.reshape(VPAD, DIM), pout.reshape(VPAD, DIM))
    loss = pl.pallas_call(
        _bce_body,
        out_shape=jax.ShapeDtypeStruct((1, 1), jnp.float32),
    )(logits.reshape(_NROWS, 128))
    return loss[0, 0]


# speedup vs baseline: 1.2067x; 1.0044x over previous
import functools

import jax
import jax.numpy as jnp
from jax import lax
from jax.experimental import pallas as pl
from jax.experimental.pallas import tpu as pltpu
from jax.experimental.pallas import tpu_sc as plsc

VOCAB = 100000
DIM = 64
B = 16384
NEG = 5
CK = 1 + NEG

TBLK = 16384
NBLK = (VOCAB + TBLK - 1) // TBLK
VPAD = NBLK * TBLK

_info = plsc.get_sparse_core_info()
NC, NS, L = _info.num_cores, _info.num_subcores, _info.num_lanes
NW = NC * NS
BPW = B // NW
CHUNK = 128
NCH = BPW // CHUNK
GROUPS = CHUNK // 16

_mesh = plsc.VectorSubcoreMesh(core_axis_name="c", subcore_axis_name="s")


def _repack_body(win_ref, wout_ref, pin_ref, pout_ref):
    r = lax.broadcasted_iota(jnp.int32, (DIM, DIM), 0)
    c = lax.broadcasted_iota(jnp.int32, (DIM, DIM), 1)
    eye = jnp.where(r == c, 1.0, 0.0).astype(jnp.bfloat16)
    dn = (((0,), (0,)), ((), ()))
    ti = lax.dot_general(win_ref[...].astype(jnp.bfloat16), eye, dn,
                         preferred_element_type=jnp.float32)
    pin_ref[:, 0:DIM] = ti[:TBLK // 2]
    pin_ref[:, DIM:128] = ti[TBLK // 2:]
    to = lax.dot_general(wout_ref[...].astype(jnp.bfloat16), eye, dn,
                         preferred_element_type=jnp.float32)
    pout_ref[:, 0:DIM] = to[:TBLK // 2]
    pout_ref[:, DIM:128] = to[TBLK // 2:]


_repack = pl.pallas_call(
    _repack_body,
    grid=(NBLK,),
    in_specs=[
        pl.BlockSpec((DIM, TBLK), lambda i: (0, i)),
        pl.BlockSpec((DIM, TBLK), lambda i: (0, i)),
    ],
    out_specs=[
        pl.BlockSpec((TBLK // 2, 128), lambda i: (i, 0)),
        pl.BlockSpec((TBLK // 2, 128), lambda i: (i, 0)),
    ],
    out_shape=[
        jax.ShapeDtypeStruct((VPAD // 2, 128), jnp.float32),
        jax.ShapeDtypeStruct((VPAD // 2, 128), jnp.float32),
    ],
)

_HSH = (TBLK // 2).bit_length() - 1


@functools.partial(
    pl.kernel,
    out_type=jax.ShapeDtypeStruct((B * CK,), jnp.float32),
    mesh=_mesh,
    compiler_params=pltpu.CompilerParams(
        needs_layout_passes=False, use_tc_tiling_on_sc=False),
    scratch_types=[
        pltpu.VMEM((B // NW // 128, 128), jnp.int32),
        pltpu.VMEM((B // NW // 128, 128), jnp.int32),
        pltpu.VMEM((B * NEG // NW // 128, 128), jnp.int32),
        pltpu.VMEM((CHUNK, DIM), jnp.float32),
        pltpu.VMEM((CHUNK, DIM), jnp.float32),
        pltpu.VMEM((CHUNK * CK, DIM), jnp.float32),
        pltpu.VMEM((CHUNK * CK, DIM), jnp.float32),
        pltpu.VMEM((BPW * CK,), jnp.float32),
        pltpu.SemaphoreType.DMA,
        pltpu.SemaphoreType.DMA,
    ],
)
def _sc_logits(cen_hbm, ctx_hbm, neg_hbm, w_in_hbm, w_out_hbm, out_hbm,
               cidx, xidx, nidx, crow0, crow1, prow0, prow1, logit_v,
               sem0, sem1):
    wid = lax.axis_index("s") * NC + lax.axis_index("c")
    crows = (crow0, crow1)
    prows = (prow0, prow1)
    sems = (sem0, sem1)

    pltpu.sync_copy(cen_hbm.at[pl.ds(wid * NCH, NCH)], cidx)
    pltpu.sync_copy(ctx_hbm.at[pl.ds(wid * NCH, NCH)], xidx)
    for k in range(NEG):
        pltpu.sync_copy(neg_hbm.at[pl.ds(k * (B // 128) + wid * NCH, NCH)],
                        nidx.at[pl.ds(k * NCH, NCH)])

    for ref, rows in ((cidx, NCH), (xidx, NCH), (nidx, NCH * NEG)):
        for r in range(rows):
            for j in range(128 // 16):
                v = ref[r, pl.ds(j * 16, 16)]
                ref[r, pl.ds(j * 16, 16)] = (
                    jnp.bitwise_and(v, -TBLK)
                    + ((v & (TBLK // 2 - 1)) << 1)
                    + ((v >> _HSH) & 1))

    def issue(ch):
        bi = ch % 2
        sem = sems[bi]
        cp = [pltpu.async_copy(w_in_hbm.at[cidx.at[ch]], crows[bi], sem),
              pltpu.async_copy(w_out_hbm.at[xidx.at[ch]],
                               prows[bi].at[pl.ds(0, CHUNK)], sem)]
        for k in range(NEG):
            cp.append(pltpu.async_copy(
                w_out_hbm.at[nidx.at[k * NCH + ch]],
                prows[bi].at[pl.ds(CHUNK + k * CHUNK, CHUNK)], sem))
        return cp

    iota16 = lax.iota(jnp.int32, 16)
    pending = issue(0)
    for ch in range(NCH):
        nxt = issue(ch + 1) if ch + 1 < NCH else []
        for c in pending:
            c.wait()
        pending = nxt
        bi = ch % 2
        crow, prow = crows[bi], prows[bi]

        def group_body(g, _, ch=ch, crow=crow, prow=prow):
            bvec = iota16 + g * 16
            rowp = [bvec] + [CHUNK + k * CHUNK + bvec for k in range(NEG)]

            def d_block(db, accs):
                out = list(accs)
                for dd in range(8):
                    dvec = jnp.bitwise_and(iota16 + (db * 8 + dd), DIM - 1)
                    cv = plsc.load_gather(crow, [bvec, dvec])
                    for p in range(CK):
                        xv = plsc.load_gather(prow, [rowp[p], dvec])
                        out[p] = out[p] + cv * xv
                return tuple(out)

            accs = lax.fori_loop(
                0, DIM // 8, d_block,
                tuple(jnp.zeros((16,), jnp.float32) for _ in range(CK)))
            flat = (bvec + ch * CHUNK) * CK
            for p in range(CK):
                plsc.store_scatter(logit_v, [flat + p], accs[p])
            return 0

        lax.fori_loop(0, GROUPS, group_body, 0)

    pltpu.sync_copy(logit_v, out_hbm.at[pl.ds(wid * BPW * CK, BPW * CK)])


_NROWS = B * CK // 128


def _bce_body(lg_ref, out_ref):
    x = lg_ref[...]
    r = lax.broadcasted_iota(jnp.int32, (_NROWS, 128), 0)
    c = lax.broadcasted_iota(jnp.int32, (_NROWS, 128), 1)
    pos = (r * 128 + c) % CK
    y = jnp.where(pos == 0, 1.0, 0.0)
    elem = jnp.maximum(x, 0.0) - x * y + jnp.log1p(jnp.exp(-jnp.abs(x)))
    out_ref[...] = (jnp.sum(elem) * (1.0 / (B * CK))).reshape(1, 1)


def kernel(center, context, negatives, W_in, W_out):
    cen = center.astype(jnp.int32).reshape(-1, 128)
    ctx = context.astype(jnp.int32).reshape(-1, 128)
    neg = negatives.T.astype(jnp.int32).reshape(-1, 128)
    pin, pout = _repack(W_in.T, W_out.T)
    logits = _sc_logits(cen, ctx, neg,
                        pin.reshape(VPAD, DIM), pout.reshape(VPAD, DIM))
    loss = pl.pallas_call(
        _bce_body,
        out_shape=jax.ShapeDtypeStruct((1, 1), jnp.float32),
    )(logits.reshape(_NROWS, 128))
    return loss[0, 0]
